# baseline (speedup 1.0000x reference)
import functools

import jax
import jax.numpy as jnp
from jax import lax
from jax.experimental import pallas as pl
from jax.experimental.pallas import tpu as pltpu
from jax.experimental.pallas import tpu_sc as plsc

_VOCAB = 1000000
_D = 64
_B = 16384
_K = 20

_NC = 2
_NS = 16
_NW = _NC * _NS
_BPW = _B // _NW
_G = 16
_NG = _BPW // _G
_GK = _G * _K
_NR = _BPW * _K // 128
_NCH = (64, 64, 64, 64, 64)


def _sc_body(tgt_hbm, ctx_hbm, negf_hbm, u_hbm, v_hbm, w_hbm,
             pos_hbm, negdot_hbm, pred_hbm,
             idx_u, idx_v, h_u, h_v, idx_n, h_n,
             u_a, v_a, n_a, u_b, v_b, n_b, w_vmem,
             pos_buf, pred_buf, neg_buf, sem_a, sem_b, sem_w):
    wid = lax.axis_index("s") * _NC + lax.axis_index("c")
    eb0 = wid * _BPW

    pltpu.async_copy(w_hbm, w_vmem, sem_w)
    pltpu.sync_copy(tgt_hbm.at[pl.ds(eb0, _BPW)], idx_u)
    pltpu.sync_copy(ctx_hbm.at[pl.ds(eb0, _BPW)], idx_v)
    pltpu.sync_copy(negf_hbm.at[pl.ds(wid * _NR, _NR)], idx_n)
    pltpu.make_async_copy(w_hbm, w_vmem, sem_w).wait()

    @pl.loop(0, _BPW // 16)
    def _split_uv(i):
        xu = idx_u[pl.ds(i * 16, 16)]
        idx_u[pl.ds(i * 16, 16)] = ((xu >> 8) << 7) + (xu & 127)
        h_u[pl.ds(i * 16, 16)] = (xu & 128) >> 1
        xv = idx_v[pl.ds(i * 16, 16)]
        idx_v[pl.ds(i * 16, 16)] = ((xv >> 8) << 7) + (xv & 127)
        h_v[pl.ds(i * 16, 16)] = (xv & 128) >> 1

    @pl.loop(0, _NR)
    def _split_n(r):
        for j in range(8):
            x = idx_n[r, pl.ds(j * 16, 16)]
            idx_n[r, pl.ds(j * 16, 16)] = ((x >> 8) << 7) + (x & 127)
            h_n[r, pl.ds(j * 16, 16)] = (x & 128) >> 1

    def issue(g, ub, vb, nb, sem):
        pltpu.async_copy(u_hbm.at[idx_u.at[pl.ds(g * _G, _G)]], ub, sem)
        pltpu.async_copy(v_hbm.at[idx_v.at[pl.ds(g * _G, _G)]], vb, sem)
        fb = g * _GK
        off = 0
        for c in _NCH:
            r, col = (fb + off) // 128, (fb + off) % 128
            pltpu.async_copy(
                v_hbm.at[idx_n.at[r, pl.ds(col, c)]],
                nb.at[pl.ds(off, c)], sem)
            off += c

    def wait_group(ub, vb, nb, sem):
        pltpu.make_async_copy(u_hbm.at[idx_u.at[pl.ds(0, _G)]], ub, sem).wait()
        pltpu.make_async_copy(v_hbm.at[idx_v.at[pl.ds(0, _G)]], vb, sem).wait()
        off = 0
        for c in _NCH:
            pltpu.make_async_copy(
                v_hbm.at[idx_n.at[0, pl.ds(0, c)]],
                nb.at[pl.ds(off, c)], sem).wait()
            off += c

    iota = jnp.arange(16, dtype=jnp.int32)

    def compute(g, ub, vb, nb):
        nrows = [iota * _K + k for k in range(_K)]
        ob = g * _G
        nob = g * _GK
        bu = h_u[pl.ds(g * _G, 16)]
        bv = h_v[pl.ds(g * _G, 16)]
        bns = []
        for k in range(_K):
            f = nob + iota * _K + k
            bns.append(plsc.load_gather(h_n, [f >> 7, f & 127]))

        init = [jnp.zeros((16,), jnp.float32)] * 7

        def dbody0(d, accs):
            dcol = (iota + d) & (_D - 1)
            u_d = plsc.load_gather(ub, [iota, bu + dcol])
            v_d = plsc.load_gather(vb, [iota, bv + dcol])
            w_d = plsc.load_gather(w_vmem, [dcol])
            out = [accs[0] + u_d * v_d, accs[1] + u_d * w_d]
            for k in range(5):
                n_d = plsc.load_gather(nb, [nrows[k], bns[k] + dcol])
                out.append(accs[2 + k] + u_d * n_d)
            return out

        accs = pl.loop(0, _D, init_carry=init, unroll=2)(dbody0)
        pos_buf[pl.ds(ob, 16)] = accs[0]
        pred_buf[pl.ds(ob, 16)] = accs[1]
        for k in range(5):
            neg_buf[pl.ds(nob + k * 16, 16)] = accs[2 + k]

        for kc in range(5, _K, 5):
            init = [jnp.zeros((16,), jnp.float32)] * 5

            def dbodyk(d, accs, _kc=kc):
                dcol = (iota + d) & (_D - 1)
                u_d = plsc.load_gather(ub, [iota, bu + dcol])
                out = []
                for k in range(5):
                    n_d = plsc.load_gather(
                        nb, [nrows[_kc + k], bns[_kc + k] + dcol])
                    out.append(accs[k] + u_d * n_d)
                return out

            accs = pl.loop(0, _D, init_carry=init, unroll=2)(dbodyk)
            for k in range(5):
                neg_buf[pl.ds(nob + (kc + k) * 16, 16)] = accs[k]

    issue(0, u_a, v_a, n_a, sem_a)

    @pl.loop(0, _NG // 2)
    def _pair(p):
        g0 = 2 * p
        issue(g0 + 1, u_b, v_b, n_b, sem_b)
        wait_group(u_a, v_a, n_a, sem_a)
        compute(g0, u_a, v_a, n_a)

        @pl.when(p < _NG // 2 - 1)
        def _():
            issue(g0 + 2, u_a, v_a, n_a, sem_a)

        wait_group(u_b, v_b, n_b, sem_b)
        compute(g0 + 1, u_b, v_b, n_b)

    pltpu.sync_copy(pos_buf, pos_hbm.at[pl.ds(eb0, _BPW)])
    pltpu.sync_copy(pred_buf, pred_hbm.at[pl.ds(eb0, _BPW)])
    pltpu.sync_copy(neg_buf, negdot_hbm.at[pl.ds(eb0 * _K, _BPW * _K)])


_sc_dots = functools.partial(
    pl.kernel,
    out_type=[
        jax.ShapeDtypeStruct((_B,), jnp.float32),
        jax.ShapeDtypeStruct((_B * _K,), jnp.float32),
        jax.ShapeDtypeStruct((_B,), jnp.float32),
    ],
    mesh=plsc.VectorSubcoreMesh(
        core_axis_name="c", subcore_axis_name="s",
        num_cores=_NC, num_subcores=_NS),
    compiler_params=pltpu.CompilerParams(
        needs_layout_passes=False, use_tc_tiling_on_sc=True),
    scratch_types=[
        pltpu.VMEM((_BPW,), jnp.int32),
        pltpu.VMEM((_BPW,), jnp.int32),
        pltpu.VMEM((_BPW,), jnp.int32),
        pltpu.VMEM((_BPW,), jnp.int32),
        pltpu.VMEM((_NR, 128), jnp.int32),
        pltpu.VMEM((_NR, 128), jnp.int32),
        pltpu.VMEM((_G, 2 * _D), jnp.float32),
        pltpu.VMEM((_G, 2 * _D), jnp.float32),
        pltpu.VMEM((_GK, 2 * _D), jnp.float32),
        pltpu.VMEM((_G, 2 * _D), jnp.float32),
        pltpu.VMEM((_G, 2 * _D), jnp.float32),
        pltpu.VMEM((_GK, 2 * _D), jnp.float32),
        pltpu.VMEM((_D,), jnp.float32),
        pltpu.VMEM((_BPW,), jnp.float32),
        pltpu.VMEM((_BPW,), jnp.float32),
        pltpu.VMEM((_BPW * _K,), jnp.float32),
        pltpu.SemaphoreType.DMA,
        pltpu.SemaphoreType.DMA,
        pltpu.SemaphoreType.DMA,
    ],
)(_sc_body)


_PCOLS = 2048
_PGRID = -(-_VOCAB // _PCOLS)
_PROWS = _PGRID * _PCOLS // 2


def _pack_body(in_ref, out_ref):
    x = in_ref[...]
    eye = jnp.eye(_D, dtype=jnp.float32)

    def t(blk):
        return lax.dot_general(
            blk, eye, (((0,), (0,)), ((), ())),
            precision=lax.Precision.HIGHEST,
            preferred_element_type=jnp.float32)

    for j in range(_PCOLS // 256):
        lo = t(x[:, j * 256:j * 256 + 128])
        hi = t(x[:, j * 256 + 128:j * 256 + 256])
        out_ref[pl.ds(j * 128, 128), :] = jnp.concatenate([lo, hi], axis=1)


def _tc_pack(table_t):
    return pl.pallas_call(
        _pack_body,
        grid=(_PGRID,),
        in_specs=[pl.BlockSpec((_D, _PCOLS), lambda b: (0, b))],
        out_specs=pl.BlockSpec((_PCOLS // 2, 128), lambda b: (b, 0)),
        out_shape=jax.ShapeDtypeStruct((_PROWS, 128), jnp.float32),
    )(table_t)


def _tc_body(pos_ref, neg_ref, pred_ref, b_ref, loss_ref, fix_ref):
    pos = jnp.clip(pos_ref[...], -10.0, 10.0)
    neg = jnp.clip(neg_ref[...], -10.0, 10.0)
    sp_pos = jnp.maximum(-pos, 0.0) + jnp.log(1.0 + jnp.exp(-jnp.abs(pos)))
    sp_neg = jnp.maximum(neg, 0.0) + jnp.log(1.0 + jnp.exp(-jnp.abs(neg)))
    total = jnp.sum(sp_pos) + jnp.sum(sp_neg)
    loss_ref[0, 0] = total / _B
    fix_ref[...] = pred_ref[...] + b_ref[0, 0]


def _tc_finish(pos2d, neg2d, pred2d, b2d):
    return pl.pallas_call(
        _tc_body,
        out_shape=[
            jax.ShapeDtypeStruct((1, 1), jnp.float32),
            jax.ShapeDtypeStruct((_B // 128, 128), jnp.float32),
        ],
        in_specs=[
            pl.BlockSpec(memory_space=pltpu.VMEM),
            pl.BlockSpec(memory_space=pltpu.VMEM),
            pl.BlockSpec(memory_space=pltpu.VMEM),
            pl.BlockSpec(memory_space=pltpu.SMEM),
        ],
        out_specs=[
            pl.BlockSpec(memory_space=pltpu.SMEM),
            pl.BlockSpec(memory_space=pltpu.VMEM),
        ],
    )(pos2d, neg2d, pred2d, b2d)


def kernel(target_word, context_words, neg_words, u_table, v_table, W_dur, b_dur):
    tgt = target_word.astype(jnp.int32)
    ctx = context_words.astype(jnp.int32)
    negf = neg_words.astype(jnp.int32).reshape(_B * _K // 128, 128)
    w = W_dur.reshape(_D)
    u2 = _tc_pack(u_table.T)
    v2 = _tc_pack(v_table.T)
    pos, negdot, pred = _sc_dots(tgt, ctx, negf, u2, v2, w)
    loss, fix = _tc_finish(
        pos.reshape(_B // 128, 128),
        negdot.reshape(_B * _K // 128, 128),
        pred.reshape(_B // 128, 128),
        b_dur.reshape(1, 1),
    )
    return loss.reshape(()), fix.reshape(_B)


# speedup vs baseline: 1.3071x; 1.3071x over previous
import functools

import jax
import jax.numpy as jnp
from jax import lax
from jax.experimental import pallas as pl
from jax.experimental.pallas import tpu as pltpu
from jax.experimental.pallas import tpu_sc as plsc

_VOCAB = 1000000
_D = 64
_B = 16384
_K = 20

_NC = 2
_NS = 16
_NW = _NC * _NS
_BPW = _B // _NW
_G = 16
_NG = _BPW // _G
_GK = _G * _K
_NR = _BPW * _K // 128
_NCH = (64, 64, 64, 64, 64)


def _sc_body(tgt_hbm, ctx_hbm, negf_hbm, u_hbm, v_hbm, w_hbm,
             pos_hbm, negdot_hbm, pred_hbm,
             idx_u, idx_v, h_u, h_v, idx_n, h_n,
             u_a, v_a, n_a, u_b, v_b, n_b, w_vmem,
             pos_buf, pred_buf, neg_buf, sem_a, sem_b, sem_w):
    wid = lax.axis_index("s") * _NC + lax.axis_index("c")
    eb0 = wid * _BPW

    pltpu.async_copy(w_hbm, w_vmem, sem_w)
    pltpu.sync_copy(tgt_hbm.at[pl.ds(eb0, _BPW)], idx_u)
    pltpu.sync_copy(ctx_hbm.at[pl.ds(eb0, _BPW)], idx_v)
    pltpu.sync_copy(negf_hbm.at[pl.ds(wid * _NR, _NR)], idx_n)
    pltpu.make_async_copy(w_hbm, w_vmem, sem_w).wait()

    @pl.loop(0, _BPW // 16)
    def _split_uv(i):
        xu = idx_u[pl.ds(i * 16, 16)]
        idx_u[pl.ds(i * 16, 16)] = ((xu >> 8) << 7) + (xu & 127)
        h_u[pl.ds(i * 16, 16)] = (xu & 128) >> 1
        xv = idx_v[pl.ds(i * 16, 16)]
        idx_v[pl.ds(i * 16, 16)] = ((xv >> 8) << 7) + (xv & 127)
        h_v[pl.ds(i * 16, 16)] = (xv & 128) >> 1

    @pl.loop(0, _NR)
    def _split_n(r):
        for j in range(8):
            x = idx_n[r, pl.ds(j * 16, 16)]
            idx_n[r, pl.ds(j * 16, 16)] = ((x >> 8) << 7) + (x & 127)
            h_n[r, pl.ds(j * 16, 16)] = (x & 128) >> 1

    def issue(g, ub, vb, nb, sem):
        pltpu.async_copy(u_hbm.at[idx_u.at[pl.ds(g * _G, _G)]], ub, sem)
        pltpu.async_copy(v_hbm.at[idx_v.at[pl.ds(g * _G, _G)]], vb, sem)
        fb = g * _GK
        off = 0
        for c in _NCH:
            r, col = (fb + off) // 128, (fb + off) % 128
            pltpu.async_copy(
                v_hbm.at[idx_n.at[r, pl.ds(col, c)]],
                nb.at[pl.ds(off, c)], sem)
            off += c

    def wait_group(ub, vb, nb, sem):
        pltpu.make_async_copy(u_hbm.at[idx_u.at[pl.ds(0, _G)]], ub, sem).wait()
        pltpu.make_async_copy(v_hbm.at[idx_v.at[pl.ds(0, _G)]], vb, sem).wait()
        off = 0
        for c in _NCH:
            pltpu.make_async_copy(
                v_hbm.at[idx_n.at[0, pl.ds(0, c)]],
                nb.at[pl.ds(off, c)], sem).wait()
            off += c

    iota = jnp.arange(16, dtype=jnp.int32)

    def compute(g, ub, vb, nb):
        nrows = [iota * _K + k for k in range(_K)]
        ob = g * _G
        nob = g * _GK
        bu = h_u[pl.ds(g * _G, 16)]
        bv = h_v[pl.ds(g * _G, 16)]
        bns = []
        for k in range(_K):
            f = nob + iota * _K + k
            bns.append(plsc.load_gather(h_n, [f >> 7, f & 127]))

        init = [jnp.zeros((16,), jnp.float32)] * 7

        def dbody0(d, accs):
            dcol = (iota + d) & (_D - 1)
            u_d = plsc.load_gather(ub, [iota, bu + dcol])
            v_d = plsc.load_gather(vb, [iota, bv + dcol])
            w_d = plsc.load_gather(w_vmem, [dcol])
            out = [accs[0] + u_d * v_d, accs[1] + u_d * w_d]
            for k in range(5):
                n_d = plsc.load_gather(nb, [nrows[k], bns[k] + dcol])
                out.append(accs[2 + k] + u_d * n_d)
            return out

        accs = pl.loop(0, _D, init_carry=init, unroll=2)(dbody0)
        pos_buf[pl.ds(ob, 16)] = accs[0]
        pred_buf[pl.ds(ob, 16)] = accs[1]
        for k in range(5):
            neg_buf[pl.ds(nob + k * 16, 16)] = accs[2 + k]

        for kc in range(5, _K, 5):
            init = [jnp.zeros((16,), jnp.float32)] * 5

            def dbodyk(d, accs, _kc=kc):
                dcol = (iota + d) & (_D - 1)
                u_d = plsc.load_gather(ub, [iota, bu + dcol])
                out = []
                for k in range(5):
                    n_d = plsc.load_gather(
                        nb, [nrows[_kc + k], bns[_kc + k] + dcol])
                    out.append(accs[k] + u_d * n_d)
                return out

            accs = pl.loop(0, _D, init_carry=init, unroll=2)(dbodyk)
            for k in range(5):
                neg_buf[pl.ds(nob + (kc + k) * 16, 16)] = accs[k]

    issue(0, u_a, v_a, n_a, sem_a)

    @pl.loop(0, _NG // 2)
    def _pair(p):
        g0 = 2 * p
        issue(g0 + 1, u_b, v_b, n_b, sem_b)
        wait_group(u_a, v_a, n_a, sem_a)
        compute(g0, u_a, v_a, n_a)

        @pl.when(p < _NG // 2 - 1)
        def _():
            issue(g0 + 2, u_a, v_a, n_a, sem_a)

        wait_group(u_b, v_b, n_b, sem_b)
        compute(g0 + 1, u_b, v_b, n_b)

    pltpu.sync_copy(pos_buf, pos_hbm.at[pl.ds(eb0, _BPW)])
    pltpu.sync_copy(pred_buf, pred_hbm.at[pl.ds(eb0, _BPW)])
    pltpu.sync_copy(neg_buf, negdot_hbm.at[pl.ds(eb0 * _K, _BPW * _K)])


_sc_dots = functools.partial(
    pl.kernel,
    out_type=[
        jax.ShapeDtypeStruct((_B,), jnp.float32),
        jax.ShapeDtypeStruct((_B * _K,), jnp.float32),
        jax.ShapeDtypeStruct((_B,), jnp.float32),
    ],
    mesh=plsc.VectorSubcoreMesh(
        core_axis_name="c", subcore_axis_name="s",
        num_cores=_NC, num_subcores=_NS),
    compiler_params=pltpu.CompilerParams(
        needs_layout_passes=False, use_tc_tiling_on_sc=True),
    scratch_types=[
        pltpu.VMEM((_BPW,), jnp.int32),
        pltpu.VMEM((_BPW,), jnp.int32),
        pltpu.VMEM((_BPW,), jnp.int32),
        pltpu.VMEM((_BPW,), jnp.int32),
        pltpu.VMEM((_NR, 128), jnp.int32),
        pltpu.VMEM((_NR, 128), jnp.int32),
        pltpu.VMEM((_G, 2 * _D), jnp.float32),
        pltpu.VMEM((_G, 2 * _D), jnp.float32),
        pltpu.VMEM((_GK, 2 * _D), jnp.float32),
        pltpu.VMEM((_G, 2 * _D), jnp.float32),
        pltpu.VMEM((_G, 2 * _D), jnp.float32),
        pltpu.VMEM((_GK, 2 * _D), jnp.float32),
        pltpu.VMEM((_D,), jnp.float32),
        pltpu.VMEM((_BPW,), jnp.float32),
        pltpu.VMEM((_BPW,), jnp.float32),
        pltpu.VMEM((_BPW * _K,), jnp.float32),
        pltpu.SemaphoreType.DMA,
        pltpu.SemaphoreType.DMA,
        pltpu.SemaphoreType.DMA,
    ],
)(_sc_body)


_PCOLS = 2048
_PGRID = -(-_VOCAB // _PCOLS)
_PROWS = _PGRID * _PCOLS // 2


def _pack_body(in_ref, out_ref):
    y = in_ref[...].T
    for j in range(_PCOLS // 256):
        lo = y[j * 256:j * 256 + 128]
        hi = y[j * 256 + 128:j * 256 + 256]
        out_ref[pl.ds(j * 128, 128), :] = jnp.concatenate([lo, hi], axis=1)


def _tc_pack(table_t):
    return pl.pallas_call(
        _pack_body,
        grid=(_PGRID,),
        in_specs=[pl.BlockSpec((_D, _PCOLS), lambda b: (0, b))],
        out_specs=pl.BlockSpec((_PCOLS // 2, 128), lambda b: (b, 0)),
        out_shape=jax.ShapeDtypeStruct((_PROWS, 128), jnp.float32),
    )(table_t)


def _tc_body(pos_ref, neg_ref, pred_ref, b_ref, loss_ref, fix_ref):
    pos = jnp.clip(pos_ref[...], -10.0, 10.0)
    neg = jnp.clip(neg_ref[...], -10.0, 10.0)
    sp_pos = jnp.maximum(-pos, 0.0) + jnp.log(1.0 + jnp.exp(-jnp.abs(pos)))
    sp_neg = jnp.maximum(neg, 0.0) + jnp.log(1.0 + jnp.exp(-jnp.abs(neg)))
    total = jnp.sum(sp_pos) + jnp.sum(sp_neg)
    loss_ref[0, 0] = total / _B
    fix_ref[...] = pred_ref[...] + b_ref[0, 0]


def _tc_finish(pos2d, neg2d, pred2d, b2d):
    return pl.pallas_call(
        _tc_body,
        out_shape=[
            jax.ShapeDtypeStruct((1, 1), jnp.float32),
            jax.ShapeDtypeStruct((_B // 128, 128), jnp.float32),
        ],
        in_specs=[
            pl.BlockSpec(memory_space=pltpu.VMEM),
            pl.BlockSpec(memory_space=pltpu.VMEM),
            pl.BlockSpec(memory_space=pltpu.VMEM),
            pl.BlockSpec(memory_space=pltpu.SMEM),
        ],
        out_specs=[
            pl.BlockSpec(memory_space=pltpu.SMEM),
            pl.BlockSpec(memory_space=pltpu.VMEM),
        ],
    )(pos2d, neg2d, pred2d, b2d)


def kernel(target_word, context_words, neg_words, u_table, v_table, W_dur, b_dur):
    tgt = target_word.astype(jnp.int32)
    ctx = context_words.astype(jnp.int32)
    negf = neg_words.astype(jnp.int32).reshape(_B * _K // 128, 128)
    w = W_dur.reshape(_D)
    u2 = _tc_pack(u_table.T)
    v2 = _tc_pack(v_table.T)
    pos, negdot, pred = _sc_dots(tgt, ctx, negf, u2, v2, w)
    loss, fix = _tc_finish(
        pos.reshape(_B // 128, 128),
        negdot.reshape(_B * _K // 128, 128),
        pred.reshape(_B // 128, 128),
        b_dur.reshape(1, 1),
    )
    return loss.reshape(()), fix.reshape(_B)


# speedup vs baseline: 1.7195x; 1.3155x over previous
import functools

import jax
import jax.numpy as jnp
from jax import lax
from jax.experimental import pallas as pl
from jax.experimental.pallas import tpu as pltpu
from jax.experimental.pallas import tpu_sc as plsc

_VOCAB = 1000000
_D = 64
_B = 16384
_K = 20

_NC = 2
_NS = 16
_NW = _NC * _NS
_BPW = _B // _NW
_G = 16
_NG = _BPW // _G
_GK = _G * _K
_NR = _BPW * _K // 128
_NCH = (64, 64, 64, 64, 64)


def _sc_body(tgt_hbm, ctx_hbm, negf_hbm, u_hbm, v_hbm, w_hbm,
             pos_hbm, negdot_hbm, pred_hbm,
             idx_u, idx_v, h_u, h_v, idx_n, h_n,
             u_a, v_a, n_a, u_b, v_b, n_b, w_vmem,
             pos_buf, pred_buf, neg_buf, sem_a, sem_b, sem_w):
    wid = lax.axis_index("s") * _NC + lax.axis_index("c")
    eb0 = wid * _BPW

    pltpu.async_copy(w_hbm, w_vmem, sem_w)
    pltpu.sync_copy(tgt_hbm.at[pl.ds(eb0, _BPW)], idx_u)
    pltpu.sync_copy(ctx_hbm.at[pl.ds(eb0, _BPW)], idx_v)
    pltpu.sync_copy(negf_hbm.at[pl.ds(wid * _NR, _NR)], idx_n)
    pltpu.make_async_copy(w_hbm, w_vmem, sem_w).wait()

    @pl.loop(0, _BPW // 16)
    def _split_uv(i):
        xu = idx_u[pl.ds(i * 16, 16)]
        idx_u[pl.ds(i * 16, 16)] = ((xu >> 8) << 7) + (xu & 127)
        h_u[pl.ds(i * 16, 16)] = (xu & 128) >> 1
        xv = idx_v[pl.ds(i * 16, 16)]
        idx_v[pl.ds(i * 16, 16)] = ((xv >> 8) << 7) + (xv & 127)
        h_v[pl.ds(i * 16, 16)] = (xv & 128) >> 1

    @pl.loop(0, _NR)
    def _split_n(r):
        for j in range(8):
            x = idx_n[r, pl.ds(j * 16, 16)]
            idx_n[r, pl.ds(j * 16, 16)] = ((x >> 8) << 7) + (x & 127)
            h_n[r, pl.ds(j * 16, 16)] = (x & 128) >> 1

    def issue(g, ub, vb, nb, sem):
        pltpu.async_copy(u_hbm.at[idx_u.at[pl.ds(g * _G, _G)]], ub, sem)
        pltpu.async_copy(v_hbm.at[idx_v.at[pl.ds(g * _G, _G)]], vb, sem)
        fb = g * _GK
        off = 0
        for c in _NCH:
            r, col = (fb + off) // 128, (fb + off) % 128
            pltpu.async_copy(
                v_hbm.at[idx_n.at[r, pl.ds(col, c)]],
                nb.at[pl.ds(off, c)], sem)
            off += c

    def wait_group(ub, vb, nb, sem):
        pltpu.make_async_copy(u_hbm.at[idx_u.at[pl.ds(0, _G)]], ub, sem).wait()
        pltpu.make_async_copy(v_hbm.at[idx_v.at[pl.ds(0, _G)]], vb, sem).wait()
        off = 0
        for c in _NCH:
            pltpu.make_async_copy(
                v_hbm.at[idx_n.at[0, pl.ds(0, c)]],
                nb.at[pl.ds(off, c)], sem).wait()
            off += c

    iota = jnp.arange(16, dtype=jnp.int32)

    def compute(g, ub, vb, nb):
        nrows = [iota * _K + k for k in range(_K)]
        ob = g * _G
        nob = g * _GK
        bu = h_u[pl.ds(g * _G, 16)]
        bv = h_v[pl.ds(g * _G, 16)]
        bns = []
        for k in range(_K):
            f = nob + iota * _K + k
            bns.append(plsc.load_gather(h_n, [f >> 7, f & 127]))

        init = [jnp.zeros((16,), jnp.float32)] * 7

        def dbody0(d, accs):
            dcol = (iota + d) & (_D - 1)
            u_d = plsc.load_gather(ub, [iota, bu + dcol])
            v_d = plsc.load_gather(vb, [iota, bv + dcol])
            w_d = plsc.load_gather(w_vmem, [dcol])
            out = [accs[0] + u_d * v_d, accs[1] + u_d * w_d]
            for k in range(5):
                n_d = plsc.load_gather(nb, [nrows[k], bns[k] + dcol])
                out.append(accs[2 + k] + u_d * n_d)
            return out

        accs = pl.loop(0, _D, init_carry=init, unroll=2)(dbody0)
        pos_buf[pl.ds(ob, 16)] = accs[0]
        pred_buf[pl.ds(ob, 16)] = accs[1]
        for k in range(5):
            neg_buf[pl.ds(nob + k * 16, 16)] = accs[2 + k]

        for kc in range(5, _K, 5):
            init = [jnp.zeros((16,), jnp.float32)] * 5

            def dbodyk(d, accs, _kc=kc):
                dcol = (iota + d) & (_D - 1)
                u_d = plsc.load_gather(ub, [iota, bu + dcol])
                out = []
                for k in range(5):
                    n_d = plsc.load_gather(
                        nb, [nrows[_kc + k], bns[_kc + k] + dcol])
                    out.append(accs[k] + u_d * n_d)
                return out

            accs = pl.loop(0, _D, init_carry=init, unroll=2)(dbodyk)
            for k in range(5):
                neg_buf[pl.ds(nob + (kc + k) * 16, 16)] = accs[k]

    issue(0, u_a, v_a, n_a, sem_a)

    @pl.loop(0, _NG // 2)
    def _pair(p):
        g0 = 2 * p
        issue(g0 + 1, u_b, v_b, n_b, sem_b)
        wait_group(u_a, v_a, n_a, sem_a)
        compute(g0, u_a, v_a, n_a)

        @pl.when(p < _NG // 2 - 1)
        def _():
            issue(g0 + 2, u_a, v_a, n_a, sem_a)

        wait_group(u_b, v_b, n_b, sem_b)
        compute(g0 + 1, u_b, v_b, n_b)

    pltpu.sync_copy(pos_buf, pos_hbm.at[pl.ds(eb0, _BPW)])
    pltpu.sync_copy(pred_buf, pred_hbm.at[pl.ds(eb0, _BPW)])
    pltpu.sync_copy(neg_buf, negdot_hbm.at[pl.ds(eb0 * _K, _BPW * _K)])


_sc_dots = functools.partial(
    pl.kernel,
    out_type=[
        jax.ShapeDtypeStruct((_B,), jnp.float32),
        jax.ShapeDtypeStruct((_B * _K,), jnp.float32),
        jax.ShapeDtypeStruct((_B,), jnp.float32),
    ],
    mesh=plsc.VectorSubcoreMesh(
        core_axis_name="c", subcore_axis_name="s",
        num_cores=_NC, num_subcores=_NS),
    compiler_params=pltpu.CompilerParams(
        needs_layout_passes=False, use_tc_tiling_on_sc=True),
    scratch_types=[
        pltpu.VMEM((_BPW,), jnp.int32),
        pltpu.VMEM((_BPW,), jnp.int32),
        pltpu.VMEM((_BPW,), jnp.int32),
        pltpu.VMEM((_BPW,), jnp.int32),
        pltpu.VMEM((_NR, 128), jnp.int32),
        pltpu.VMEM((_NR, 128), jnp.int32),
        pltpu.VMEM((_G, 2 * _D), jnp.float32),
        pltpu.VMEM((_G, 2 * _D), jnp.float32),
        pltpu.VMEM((_GK, 2 * _D), jnp.float32),
        pltpu.VMEM((_G, 2 * _D), jnp.float32),
        pltpu.VMEM((_G, 2 * _D), jnp.float32),
        pltpu.VMEM((_GK, 2 * _D), jnp.float32),
        pltpu.VMEM((_D,), jnp.float32),
        pltpu.VMEM((_BPW,), jnp.float32),
        pltpu.VMEM((_BPW,), jnp.float32),
        pltpu.VMEM((_BPW * _K,), jnp.float32),
        pltpu.SemaphoreType.DMA,
        pltpu.SemaphoreType.DMA,
        pltpu.SemaphoreType.DMA,
    ],
)(_sc_body)


_PCOLS = 4096
_PGRID = -(-_VOCAB // _PCOLS)
_PROWS = _PGRID * _PCOLS // 2


def _pack_body(in_ref, out_ref):
    y = in_ref[...].T
    for j in range(_PCOLS // 256):
        lo = y[j * 256:j * 256 + 128]
        hi = y[j * 256 + 128:j * 256 + 256]
        out_ref[pl.ds(j * 128, 128), :] = jnp.concatenate([lo, hi], axis=1)


def _tc_pack(table_t):
    return pl.pallas_call(
        _pack_body,
        grid=(_PGRID,),
        in_specs=[pl.BlockSpec((_D, _PCOLS), lambda b: (0, b))],
        out_specs=pl.BlockSpec((_PCOLS // 2, 128), lambda b: (b, 0)),
        out_shape=jax.ShapeDtypeStruct((_PROWS, 128), jnp.float32),
    )(table_t)


def _tc_body(pos_ref, neg_ref, pred_ref, b_ref, loss_ref, fix_ref):
    pos = jnp.clip(pos_ref[...], -10.0, 10.0)
    neg = jnp.clip(neg_ref[...], -10.0, 10.0)
    sp_pos = jnp.maximum(-pos, 0.0) + jnp.log(1.0 + jnp.exp(-jnp.abs(pos)))
    sp_neg = jnp.maximum(neg, 0.0) + jnp.log(1.0 + jnp.exp(-jnp.abs(neg)))
    total = jnp.sum(sp_pos) + jnp.sum(sp_neg)
    loss_ref[0, 0] = total / _B
    fix_ref[...] = pred_ref[...] + b_ref[0, 0]


def _tc_finish(pos2d, neg2d, pred2d, b2d):
    return pl.pallas_call(
        _tc_body,
        out_shape=[
            jax.ShapeDtypeStruct((1, 1), jnp.float32),
            jax.ShapeDtypeStruct((_B // 128, 128), jnp.float32),
        ],
        in_specs=[
            pl.BlockSpec(memory_space=pltpu.VMEM),
            pl.BlockSpec(memory_space=pltpu.VMEM),
            pl.BlockSpec(memory_space=pltpu.VMEM),
            pl.BlockSpec(memory_space=pltpu.SMEM),
        ],
        out_specs=[
            pl.BlockSpec(memory_space=pltpu.SMEM),
            pl.BlockSpec(memory_space=pltpu.VMEM),
        ],
    )(pos2d, neg2d, pred2d, b2d)


def kernel(target_word, context_words, neg_words, u_table, v_table, W_dur, b_dur):
    tgt = target_word.astype(jnp.int32)
    ctx = context_words.astype(jnp.int32)
    negf = neg_words.astype(jnp.int32).reshape(_B * _K // 128, 128)
    w = W_dur.reshape(_D)
    u2 = _tc_pack(u_table.T)
    v2 = _tc_pack(v_table.T)
    pos, negdot, pred = _sc_dots(tgt, ctx, negf, u2, v2, w)
    loss, fix = _tc_finish(
        pos.reshape(_B // 128, 128),
        negdot.reshape(_B * _K // 128, 128),
        pred.reshape(_B // 128, 128),
        b_dur.reshape(1, 1),
    )
    return loss.reshape(()), fix.reshape(_B)


# speedup vs baseline: 2.0782x; 1.2086x over previous
import functools

import jax
import jax.numpy as jnp
from jax import lax
from jax.experimental import pallas as pl
from jax.experimental.pallas import tpu as pltpu
from jax.experimental.pallas import tpu_sc as plsc

_VOCAB = 1000000
_D = 64
_B = 16384
_K = 20

_NC = 2
_NS = 16
_NW = _NC * _NS
_BPW = _B // _NW
_G = 16
_NG = _BPW // _G
_GK = _G * _K
_NR = _BPW * _K // 128
_NCH = (64, 64, 64, 64, 64)


def _sc_body(tgt_hbm, ctx_hbm, negf_hbm, u_hbm, v_hbm, w_hbm,
             pos_hbm, negdot_hbm, pred_hbm,
             idx_u, idx_v, h_u, h_v, idx_n, h_n,
             u_a, v_a, n_a, u_b, v_b, n_b, w_vmem,
             pos_buf, pred_buf, neg_buf, sem_a, sem_b, sem_w):
    wid = lax.axis_index("s") * _NC + lax.axis_index("c")
    eb0 = wid * _BPW

    pltpu.async_copy(w_hbm, w_vmem, sem_w)
    pltpu.sync_copy(tgt_hbm.at[pl.ds(eb0, _BPW)], idx_u)
    pltpu.sync_copy(ctx_hbm.at[pl.ds(eb0, _BPW)], idx_v)
    pltpu.sync_copy(negf_hbm.at[pl.ds(wid * _NR, _NR)], idx_n)
    pltpu.make_async_copy(w_hbm, w_vmem, sem_w).wait()

    @pl.loop(0, _BPW // 16)
    def _split_uv(i):
        xu = idx_u[pl.ds(i * 16, 16)]
        idx_u[pl.ds(i * 16, 16)] = ((xu >> 8) << 7) + (xu & 127)
        h_u[pl.ds(i * 16, 16)] = (xu & 128) >> 1
        xv = idx_v[pl.ds(i * 16, 16)]
        idx_v[pl.ds(i * 16, 16)] = ((xv >> 8) << 7) + (xv & 127)
        h_v[pl.ds(i * 16, 16)] = (xv & 128) >> 1

    @pl.loop(0, _NR)
    def _split_n(r):
        for j in range(8):
            x = idx_n[r, pl.ds(j * 16, 16)]
            idx_n[r, pl.ds(j * 16, 16)] = ((x >> 8) << 7) + (x & 127)
            h_n[r, pl.ds(j * 16, 16)] = (x & 128) >> 1

    def issue(g, ub, vb, nb, sem):
        pltpu.async_copy(u_hbm.at[idx_u.at[pl.ds(g * _G, _G)]], ub, sem)
        pltpu.async_copy(v_hbm.at[idx_v.at[pl.ds(g * _G, _G)]], vb, sem)
        fb = g * _GK
        off = 0
        for c in _NCH:
            r, col = (fb + off) // 128, (fb + off) % 128
            pltpu.async_copy(
                v_hbm.at[idx_n.at[r, pl.ds(col, c)]],
                nb.at[pl.ds(off, c)], sem)
            off += c

    def wait_group(ub, vb, nb, sem):
        pltpu.make_async_copy(u_hbm.at[idx_u.at[pl.ds(0, _G)]], ub, sem).wait()
        pltpu.make_async_copy(v_hbm.at[idx_v.at[pl.ds(0, _G)]], vb, sem).wait()
        off = 0
        for c in _NCH:
            pltpu.make_async_copy(
                v_hbm.at[idx_n.at[0, pl.ds(0, c)]],
                nb.at[pl.ds(off, c)], sem).wait()
            off += c

    iota = jnp.arange(16, dtype=jnp.int32)

    def compute(g, ub, vb, nb):
        nrows = [iota * _K + k for k in range(_K)]
        ob = g * _G
        nob = g * _GK
        bu = h_u[pl.ds(g * _G, 16)]
        bv = h_v[pl.ds(g * _G, 16)]
        bns = []
        for k in range(_K):
            f = nob + iota * _K + k
            bns.append(plsc.load_gather(h_n, [f >> 7, f & 127]))

        init = [jnp.zeros((16,), jnp.float32)] * 7

        def dbody0(d, accs):
            dcol = (iota + d) & (_D - 1)
            u_d = plsc.load_gather(ub, [iota, bu + dcol])
            v_d = plsc.load_gather(vb, [iota, bv + dcol])
            w_d = plsc.load_gather(w_vmem, [dcol])
            out = [accs[0] + u_d * v_d, accs[1] + u_d * w_d]
            for k in range(5):
                n_d = plsc.load_gather(nb, [nrows[k], bns[k] + dcol])
                out.append(accs[2 + k] + u_d * n_d)
            return out

        accs = pl.loop(0, _D, init_carry=init, unroll=2)(dbody0)
        pos_buf[pl.ds(ob, 16)] = accs[0]
        pred_buf[pl.ds(ob, 16)] = accs[1]
        for k in range(5):
            neg_buf[pl.ds(nob + k * 16, 16)] = accs[2 + k]

        for kc in range(5, _K, 5):
            init = [jnp.zeros((16,), jnp.float32)] * 5

            def dbodyk(d, accs, _kc=kc):
                dcol = (iota + d) & (_D - 1)
                u_d = plsc.load_gather(ub, [iota, bu + dcol])
                out = []
                for k in range(5):
                    n_d = plsc.load_gather(
                        nb, [nrows[_kc + k], bns[_kc + k] + dcol])
                    out.append(accs[k] + u_d * n_d)
                return out

            accs = pl.loop(0, _D, init_carry=init, unroll=2)(dbodyk)
            for k in range(5):
                neg_buf[pl.ds(nob + (kc + k) * 16, 16)] = accs[k]

    issue(0, u_a, v_a, n_a, sem_a)

    @pl.loop(0, _NG // 2)
    def _pair(p):
        g0 = 2 * p
        issue(g0 + 1, u_b, v_b, n_b, sem_b)
        wait_group(u_a, v_a, n_a, sem_a)
        compute(g0, u_a, v_a, n_a)

        @pl.when(p < _NG // 2 - 1)
        def _():
            issue(g0 + 2, u_a, v_a, n_a, sem_a)

        wait_group(u_b, v_b, n_b, sem_b)
        compute(g0 + 1, u_b, v_b, n_b)

    pltpu.sync_copy(pos_buf, pos_hbm.at[pl.ds(eb0, _BPW)])
    pltpu.sync_copy(pred_buf, pred_hbm.at[pl.ds(eb0, _BPW)])
    pltpu.sync_copy(neg_buf, negdot_hbm.at[pl.ds(eb0 * _K, _BPW * _K)])


_sc_dots = functools.partial(
    pl.kernel,
    out_type=[
        jax.ShapeDtypeStruct((_B,), jnp.float32),
        jax.ShapeDtypeStruct((_B * _K,), jnp.float32),
        jax.ShapeDtypeStruct((_B,), jnp.float32),
    ],
    mesh=plsc.VectorSubcoreMesh(
        core_axis_name="c", subcore_axis_name="s",
        num_cores=_NC, num_subcores=_NS),
    compiler_params=pltpu.CompilerParams(
        needs_layout_passes=False, use_tc_tiling_on_sc=True),
    scratch_types=[
        pltpu.VMEM((_BPW,), jnp.int32),
        pltpu.VMEM((_BPW,), jnp.int32),
        pltpu.VMEM((_BPW,), jnp.int32),
        pltpu.VMEM((_BPW,), jnp.int32),
        pltpu.VMEM((_NR, 128), jnp.int32),
        pltpu.VMEM((_NR, 128), jnp.int32),
        pltpu.VMEM((_G, 2 * _D), jnp.float32),
        pltpu.VMEM((_G, 2 * _D), jnp.float32),
        pltpu.VMEM((_GK, 2 * _D), jnp.float32),
        pltpu.VMEM((_G, 2 * _D), jnp.float32),
        pltpu.VMEM((_G, 2 * _D), jnp.float32),
        pltpu.VMEM((_GK, 2 * _D), jnp.float32),
        pltpu.VMEM((_D,), jnp.float32),
        pltpu.VMEM((_BPW,), jnp.float32),
        pltpu.VMEM((_BPW,), jnp.float32),
        pltpu.VMEM((_BPW * _K,), jnp.float32),
        pltpu.SemaphoreType.DMA,
        pltpu.SemaphoreType.DMA,
        pltpu.SemaphoreType.DMA,
    ],
)(_sc_body)


_PCOLS = 8192
_PGRID = -(-_VOCAB // _PCOLS)
_PROWS = _PGRID * _PCOLS // 2


def _pack_body(in_ref, out_ref):
    y = in_ref[...].T
    for j in range(_PCOLS // 256):
        lo = y[j * 256:j * 256 + 128]
        hi = y[j * 256 + 128:j * 256 + 256]
        out_ref[pl.ds(j * 128, 128), :] = jnp.concatenate([lo, hi], axis=1)


def _tc_pack(table_t):
    return pl.pallas_call(
        _pack_body,
        grid=(_PGRID,),
        in_specs=[pl.BlockSpec((_D, _PCOLS), lambda b: (0, b))],
        out_specs=pl.BlockSpec((_PCOLS // 2, 128), lambda b: (b, 0)),
        out_shape=jax.ShapeDtypeStruct((_PROWS, 128), jnp.float32),
    )(table_t)


def _tc_body(pos_ref, neg_ref, pred_ref, b_ref, loss_ref, fix_ref):
    pos = jnp.clip(pos_ref[...], -10.0, 10.0)
    neg = jnp.clip(neg_ref[...], -10.0, 10.0)
    sp_pos = jnp.maximum(-pos, 0.0) + jnp.log(1.0 + jnp.exp(-jnp.abs(pos)))
    sp_neg = jnp.maximum(neg, 0.0) + jnp.log(1.0 + jnp.exp(-jnp.abs(neg)))
    total = jnp.sum(sp_pos) + jnp.sum(sp_neg)
    loss_ref[0, 0] = total / _B
    fix_ref[...] = pred_ref[...] + b_ref[0, 0]


def _tc_finish(pos2d, neg2d, pred2d, b2d):
    return pl.pallas_call(
        _tc_body,
        out_shape=[
            jax.ShapeDtypeStruct((1, 1), jnp.float32),
            jax.ShapeDtypeStruct((_B // 128, 128), jnp.float32),
        ],
        in_specs=[
            pl.BlockSpec(memory_space=pltpu.VMEM),
            pl.BlockSpec(memory_space=pltpu.VMEM),
            pl.BlockSpec(memory_space=pltpu.VMEM),
            pl.BlockSpec(memory_space=pltpu.SMEM),
        ],
        out_specs=[
            pl.BlockSpec(memory_space=pltpu.SMEM),
            pl.BlockSpec(memory_space=pltpu.VMEM),
        ],
    )(pos2d, neg2d, pred2d, b2d)


def kernel(target_word, context_words, neg_words, u_table, v_table, W_dur, b_dur):
    tgt = target_word.astype(jnp.int32)
    ctx = context_words.astype(jnp.int32)
    negf = neg_words.astype(jnp.int32).reshape(_B * _K // 128, 128)
    w = W_dur.reshape(_D)
    u2 = _tc_pack(u_table.T)
    v2 = _tc_pack(v_table.T)
    pos, negdot, pred = _sc_dots(tgt, ctx, negf, u2, v2, w)
    loss, fix = _tc_finish(
        pos.reshape(_B // 128, 128),
        negdot.reshape(_B * _K // 128, 128),
        pred.reshape(_B // 128, 128),
        b_dur.reshape(1, 1),
    )
    return loss.reshape(()), fix.reshape(_B)


# speedup vs baseline: 2.3228x; 1.1177x over previous
import functools

import jax
import jax.numpy as jnp
from jax import lax
from jax.experimental import pallas as pl
from jax.experimental.pallas import tpu as pltpu
from jax.experimental.pallas import tpu_sc as plsc

_VOCAB = 1000000
_D = 64
_B = 16384
_K = 20

_NC = 2
_NS = 16
_NW = _NC * _NS
_BPW = _B // _NW
_G = 16
_NG = _BPW // _G
_GK = _G * _K
_NR = _BPW * _K // 128
_NCH = (64, 64, 64, 64, 64)


def _sc_body(tgt_hbm, ctx_hbm, negf_hbm, u_hbm, v_hbm, w_hbm,
             pos_hbm, negdot_hbm, pred_hbm,
             idx_u, idx_v, h_u, h_v, idx_n, h_n,
             u_a, v_a, n_a, u_b, v_b, n_b, w_vmem,
             pos_buf, pred_buf, neg_buf, sem_a, sem_b, sem_w):
    wid = lax.axis_index("s") * _NC + lax.axis_index("c")
    eb0 = wid * _BPW

    pltpu.async_copy(w_hbm, w_vmem, sem_w)
    pltpu.sync_copy(tgt_hbm.at[pl.ds(eb0, _BPW)], idx_u)
    pltpu.sync_copy(ctx_hbm.at[pl.ds(eb0, _BPW)], idx_v)
    pltpu.sync_copy(negf_hbm.at[pl.ds(wid * _NR, _NR)], idx_n)
    pltpu.make_async_copy(w_hbm, w_vmem, sem_w).wait()

    @pl.loop(0, _BPW // 16)
    def _split_uv(i):
        xu = idx_u[pl.ds(i * 16, 16)]
        idx_u[pl.ds(i * 16, 16)] = ((xu >> 8) << 7) + (xu & 127)
        h_u[pl.ds(i * 16, 16)] = (xu & 128) >> 1
        xv = idx_v[pl.ds(i * 16, 16)]
        idx_v[pl.ds(i * 16, 16)] = ((xv >> 8) << 7) + (xv & 127)
        h_v[pl.ds(i * 16, 16)] = (xv & 128) >> 1

    @pl.loop(0, _NR)
    def _split_n(r):
        for j in range(8):
            x = idx_n[r, pl.ds(j * 16, 16)]
            idx_n[r, pl.ds(j * 16, 16)] = ((x >> 8) << 7) + (x & 127)
            h_n[r, pl.ds(j * 16, 16)] = (x & 128) >> 1

    def issue(g, ub, vb, nb, sem):
        pltpu.async_copy(u_hbm.at[idx_u.at[pl.ds(g * _G, _G)]], ub, sem)
        pltpu.async_copy(v_hbm.at[idx_v.at[pl.ds(g * _G, _G)]], vb, sem)
        fb = g * _GK
        off = 0
        for c in _NCH:
            r, col = (fb + off) // 128, (fb + off) % 128
            pltpu.async_copy(
                v_hbm.at[idx_n.at[r, pl.ds(col, c)]],
                nb.at[pl.ds(off, c)], sem)
            off += c

    def wait_group(ub, vb, nb, sem):
        pltpu.make_async_copy(u_hbm.at[idx_u.at[pl.ds(0, _G)]], ub, sem).wait()
        pltpu.make_async_copy(v_hbm.at[idx_v.at[pl.ds(0, _G)]], vb, sem).wait()
        off = 0
        for c in _NCH:
            pltpu.make_async_copy(
                v_hbm.at[idx_n.at[0, pl.ds(0, c)]],
                nb.at[pl.ds(off, c)], sem).wait()
            off += c

    iota = jnp.arange(16, dtype=jnp.int32)

    def compute(g, ub, vb, nb):
        nrows = [iota * _K + k for k in range(_K)]
        ob = g * _G
        nob = g * _GK
        bu = h_u[pl.ds(g * _G, 16)]
        bv = h_v[pl.ds(g * _G, 16)]
        bns = []
        for k in range(_K):
            f = nob + iota * _K + k
            bns.append(plsc.load_gather(h_n, [f >> 7, f & 127]))

        init = [jnp.zeros((16,), jnp.float32)] * 7

        def dbody0(d, accs):
            dcol = (iota + d) & (_D - 1)
            u_d = plsc.load_gather(ub, [iota, bu + dcol])
            v_d = plsc.load_gather(vb, [iota, bv + dcol])
            w_d = plsc.load_gather(w_vmem, [dcol])
            out = [accs[0] + u_d * v_d, accs[1] + u_d * w_d]
            for k in range(5):
                n_d = plsc.load_gather(nb, [nrows[k], bns[k] + dcol])
                out.append(accs[2 + k] + u_d * n_d)
            return out

        accs = pl.loop(0, _D, init_carry=init, unroll=2)(dbody0)
        pos_buf[pl.ds(ob, 16)] = accs[0]
        pred_buf[pl.ds(ob, 16)] = accs[1]
        for k in range(5):
            neg_buf[pl.ds(nob + k * 16, 16)] = accs[2 + k]

        for kc in range(5, _K, 5):
            init = [jnp.zeros((16,), jnp.float32)] * 5

            def dbodyk(d, accs, _kc=kc):
                dcol = (iota + d) & (_D - 1)
                u_d = plsc.load_gather(ub, [iota, bu + dcol])
                out = []
                for k in range(5):
                    n_d = plsc.load_gather(
                        nb, [nrows[_kc + k], bns[_kc + k] + dcol])
                    out.append(accs[k] + u_d * n_d)
                return out

            accs = pl.loop(0, _D, init_carry=init, unroll=2)(dbodyk)
            for k in range(5):
                neg_buf[pl.ds(nob + (kc + k) * 16, 16)] = accs[k]

    issue(0, u_a, v_a, n_a, sem_a)

    @pl.loop(0, _NG // 2)
    def _pair(p):
        g0 = 2 * p
        issue(g0 + 1, u_b, v_b, n_b, sem_b)
        wait_group(u_a, v_a, n_a, sem_a)
        compute(g0, u_a, v_a, n_a)

        @pl.when(p < _NG // 2 - 1)
        def _():
            issue(g0 + 2, u_a, v_a, n_a, sem_a)

        wait_group(u_b, v_b, n_b, sem_b)
        compute(g0 + 1, u_b, v_b, n_b)

    pltpu.sync_copy(pos_buf, pos_hbm.at[pl.ds(eb0, _BPW)])
    pltpu.sync_copy(pred_buf, pred_hbm.at[pl.ds(eb0, _BPW)])
    pltpu.sync_copy(neg_buf, negdot_hbm.at[pl.ds(eb0 * _K, _BPW * _K)])


_sc_dots = functools.partial(
    pl.kernel,
    out_type=[
        jax.ShapeDtypeStruct((_B,), jnp.float32),
        jax.ShapeDtypeStruct((_B * _K,), jnp.float32),
        jax.ShapeDtypeStruct((_B,), jnp.float32),
    ],
    mesh=plsc.VectorSubcoreMesh(
        core_axis_name="c", subcore_axis_name="s",
        num_cores=_NC, num_subcores=_NS),
    compiler_params=pltpu.CompilerParams(
        needs_layout_passes=False, use_tc_tiling_on_sc=True),
    scratch_types=[
        pltpu.VMEM((_BPW,), jnp.int32),
        pltpu.VMEM((_BPW,), jnp.int32),
        pltpu.VMEM((_BPW,), jnp.int32),
        pltpu.VMEM((_BPW,), jnp.int32),
        pltpu.VMEM((_NR, 128), jnp.int32),
        pltpu.VMEM((_NR, 128), jnp.int32),
        pltpu.VMEM((_G, 2 * _D), jnp.float32),
        pltpu.VMEM((_G, 2 * _D), jnp.float32),
        pltpu.VMEM((_GK, 2 * _D), jnp.float32),
        pltpu.VMEM((_G, 2 * _D), jnp.float32),
        pltpu.VMEM((_G, 2 * _D), jnp.float32),
        pltpu.VMEM((_GK, 2 * _D), jnp.float32),
        pltpu.VMEM((_D,), jnp.float32),
        pltpu.VMEM((_BPW,), jnp.float32),
        pltpu.VMEM((_BPW,), jnp.float32),
        pltpu.VMEM((_BPW * _K,), jnp.float32),
        pltpu.SemaphoreType.DMA,
        pltpu.SemaphoreType.DMA,
        pltpu.SemaphoreType.DMA,
    ],
)(_sc_body)


_PCOLS = 16384
_PGRID = -(-_VOCAB // _PCOLS)
_PROWS = _PGRID * _PCOLS // 2


def _pack_body(in_ref, out_ref):
    y = in_ref[...].T
    for j in range(_PCOLS // 256):
        lo = y[j * 256:j * 256 + 128]
        hi = y[j * 256 + 128:j * 256 + 256]
        out_ref[pl.ds(j * 128, 128), :] = jnp.concatenate([lo, hi], axis=1)


def _tc_pack(table_t):
    return pl.pallas_call(
        _pack_body,
        grid=(_PGRID,),
        in_specs=[pl.BlockSpec((_D, _PCOLS), lambda b: (0, b))],
        out_specs=pl.BlockSpec((_PCOLS // 2, 128), lambda b: (b, 0)),
        out_shape=jax.ShapeDtypeStruct((_PROWS, 128), jnp.float32),
    )(table_t)


def _tc_body(pos_ref, neg_ref, pred_ref, b_ref, loss_ref, fix_ref):
    pos = jnp.clip(pos_ref[...], -10.0, 10.0)
    neg = jnp.clip(neg_ref[...], -10.0, 10.0)
    sp_pos = jnp.maximum(-pos, 0.0) + jnp.log(1.0 + jnp.exp(-jnp.abs(pos)))
    sp_neg = jnp.maximum(neg, 0.0) + jnp.log(1.0 + jnp.exp(-jnp.abs(neg)))
    total = jnp.sum(sp_pos) + jnp.sum(sp_neg)
    loss_ref[0, 0] = total / _B
    fix_ref[...] = pred_ref[...] + b_ref[0, 0]


def _tc_finish(pos2d, neg2d, pred2d, b2d):
    return pl.pallas_call(
        _tc_body,
        out_shape=[
            jax.ShapeDtypeStruct((1, 1), jnp.float32),
            jax.ShapeDtypeStruct((_B // 128, 128), jnp.float32),
        ],
        in_specs=[
            pl.BlockSpec(memory_space=pltpu.VMEM),
            pl.BlockSpec(memory_space=pltpu.VMEM),
            pl.BlockSpec(memory_space=pltpu.VMEM),
            pl.BlockSpec(memory_space=pltpu.SMEM),
        ],
        out_specs=[
            pl.BlockSpec(memory_space=pltpu.SMEM),
            pl.BlockSpec(memory_space=pltpu.VMEM),
        ],
    )(pos2d, neg2d, pred2d, b2d)


def kernel(target_word, context_words, neg_words, u_table, v_table, W_dur, b_dur):
    tgt = target_word.astype(jnp.int32)
    ctx = context_words.astype(jnp.int32)
    negf = neg_words.astype(jnp.int32).reshape(_B * _K // 128, 128)
    w = W_dur.reshape(_D)
    u2 = _tc_pack(u_table.T)
    v2 = _tc_pack(v_table.T)
    pos, negdot, pred = _sc_dots(tgt, ctx, negf, u2, v2, w)
    loss, fix = _tc_finish(
        pos.reshape(_B // 128, 128),
        negdot.reshape(_B * _K // 128, 128),
        pred.reshape(_B // 128, 128),
        b_dur.reshape(1, 1),
    )
    return loss.reshape(()), fix.reshape(_B)


# speedup vs baseline: 2.4508x; 1.0551x over previous
import functools

import jax
import jax.numpy as jnp
from jax import lax
from jax.experimental import pallas as pl
from jax.experimental.pallas import tpu as pltpu
from jax.experimental.pallas import tpu_sc as plsc

_VOCAB = 1000000
_D = 64
_B = 16384
_K = 20

_NC = 2
_NS = 16
_NW = _NC * _NS
_BPW = _B // _NW
_G = 16
_NG = _BPW // _G
_GK = _G * _K
_NR = _BPW * _K // 128
_NCH = (64, 64, 64, 64, 64)


def _sc_body(tgt_hbm, ctx_hbm, negf_hbm, u_hbm, v_hbm, w_hbm,
             pos_hbm, negdot_hbm, pred_hbm,
             idx_u, idx_v, h_u, h_v, idx_n, h_n,
             u_a, v_a, n_a, u_b, v_b, n_b, w_vmem,
             pos_buf, pred_buf, neg_buf, sem_a, sem_b, sem_w):
    wid = lax.axis_index("s") * _NC + lax.axis_index("c")
    eb0 = wid * _BPW

    pltpu.async_copy(w_hbm, w_vmem, sem_w)
    pltpu.sync_copy(tgt_hbm.at[pl.ds(eb0, _BPW)], idx_u)
    pltpu.sync_copy(ctx_hbm.at[pl.ds(eb0, _BPW)], idx_v)
    pltpu.sync_copy(negf_hbm.at[pl.ds(wid * _NR, _NR)], idx_n)
    pltpu.make_async_copy(w_hbm, w_vmem, sem_w).wait()

    @pl.loop(0, _BPW // 16)
    def _split_uv(i):
        xu = idx_u[pl.ds(i * 16, 16)]
        idx_u[pl.ds(i * 16, 16)] = ((xu >> 8) << 7) + (xu & 127)
        h_u[pl.ds(i * 16, 16)] = (xu & 128) >> 1
        xv = idx_v[pl.ds(i * 16, 16)]
        idx_v[pl.ds(i * 16, 16)] = ((xv >> 8) << 7) + (xv & 127)
        h_v[pl.ds(i * 16, 16)] = (xv & 128) >> 1

    @pl.loop(0, _NR)
    def _split_n(r):
        for j in range(8):
            x = idx_n[r, pl.ds(j * 16, 16)]
            idx_n[r, pl.ds(j * 16, 16)] = ((x >> 8) << 7) + (x & 127)
            h_n[r, pl.ds(j * 16, 16)] = (x & 128) >> 1

    def issue(g, ub, vb, nb, sem):
        pltpu.async_copy(u_hbm.at[idx_u.at[pl.ds(g * _G, _G)]], ub, sem)
        pltpu.async_copy(v_hbm.at[idx_v.at[pl.ds(g * _G, _G)]], vb, sem)
        fb = g * _GK
        off = 0
        for c in _NCH:
            r, col = (fb + off) // 128, (fb + off) % 128
            pltpu.async_copy(
                v_hbm.at[idx_n.at[r, pl.ds(col, c)]],
                nb.at[pl.ds(off, c)], sem)
            off += c

    def wait_group(ub, vb, nb, sem):
        pltpu.make_async_copy(u_hbm.at[idx_u.at[pl.ds(0, _G)]], ub, sem).wait()
        pltpu.make_async_copy(v_hbm.at[idx_v.at[pl.ds(0, _G)]], vb, sem).wait()
        off = 0
        for c in _NCH:
            pltpu.make_async_copy(
                v_hbm.at[idx_n.at[0, pl.ds(0, c)]],
                nb.at[pl.ds(off, c)], sem).wait()
            off += c

    iota = jnp.arange(16, dtype=jnp.int32)

    def compute(g, ub, vb, nb):
        nrows = [iota * _K + k for k in range(_K)]
        ob = g * _G
        nob = g * _GK
        bu = h_u[pl.ds(g * _G, 16)]
        bv = h_v[pl.ds(g * _G, 16)]
        bns = []
        for k in range(_K):
            f = nob + iota * _K + k
            bns.append(plsc.load_gather(h_n, [f >> 7, f & 127]))

        init = [jnp.zeros((16,), jnp.float32)] * 7

        def dbody0(d, accs):
            dcol = (iota + d) & (_D - 1)
            u_d = plsc.load_gather(ub, [iota, bu + dcol])
            v_d = plsc.load_gather(vb, [iota, bv + dcol])
            w_d = plsc.load_gather(w_vmem, [dcol])
            out = [accs[0] + u_d * v_d, accs[1] + u_d * w_d]
            for k in range(5):
                n_d = plsc.load_gather(nb, [nrows[k], bns[k] + dcol])
                out.append(accs[2 + k] + u_d * n_d)
            return out

        accs = pl.loop(0, _D, init_carry=init, unroll=2)(dbody0)
        pos_buf[pl.ds(ob, 16)] = accs[0]
        pred_buf[pl.ds(ob, 16)] = accs[1]
        for k in range(5):
            neg_buf[pl.ds(nob + k * 16, 16)] = accs[2 + k]

        for kc in range(5, _K, 5):
            init = [jnp.zeros((16,), jnp.float32)] * 5

            def dbodyk(d, accs, _kc=kc):
                dcol = (iota + d) & (_D - 1)
                u_d = plsc.load_gather(ub, [iota, bu + dcol])
                out = []
                for k in range(5):
                    n_d = plsc.load_gather(
                        nb, [nrows[_kc + k], bns[_kc + k] + dcol])
                    out.append(accs[k] + u_d * n_d)
                return out

            accs = pl.loop(0, _D, init_carry=init, unroll=2)(dbodyk)
            for k in range(5):
                neg_buf[pl.ds(nob + (kc + k) * 16, 16)] = accs[k]

    issue(0, u_a, v_a, n_a, sem_a)

    @pl.loop(0, _NG // 2)
    def _pair(p):
        g0 = 2 * p
        issue(g0 + 1, u_b, v_b, n_b, sem_b)
        wait_group(u_a, v_a, n_a, sem_a)
        compute(g0, u_a, v_a, n_a)

        @pl.when(p < _NG // 2 - 1)
        def _():
            issue(g0 + 2, u_a, v_a, n_a, sem_a)

        wait_group(u_b, v_b, n_b, sem_b)
        compute(g0 + 1, u_b, v_b, n_b)

    pltpu.sync_copy(pos_buf, pos_hbm.at[pl.ds(eb0, _BPW)])
    pltpu.sync_copy(pred_buf, pred_hbm.at[pl.ds(eb0, _BPW)])
    pltpu.sync_copy(neg_buf, negdot_hbm.at[pl.ds(eb0 * _K, _BPW * _K)])


_sc_dots = functools.partial(
    pl.kernel,
    out_type=[
        jax.ShapeDtypeStruct((_B,), jnp.float32),
        jax.ShapeDtypeStruct((_B * _K,), jnp.float32),
        jax.ShapeDtypeStruct((_B,), jnp.float32),
    ],
    mesh=plsc.VectorSubcoreMesh(
        core_axis_name="c", subcore_axis_name="s",
        num_cores=_NC, num_subcores=_NS),
    compiler_params=pltpu.CompilerParams(
        needs_layout_passes=False, use_tc_tiling_on_sc=True),
    scratch_types=[
        pltpu.VMEM((_BPW,), jnp.int32),
        pltpu.VMEM((_BPW,), jnp.int32),
        pltpu.VMEM((_BPW,), jnp.int32),
        pltpu.VMEM((_BPW,), jnp.int32),
        pltpu.VMEM((_NR, 128), jnp.int32),
        pltpu.VMEM((_NR, 128), jnp.int32),
        pltpu.VMEM((_G, 2 * _D), jnp.float32),
        pltpu.VMEM((_G, 2 * _D), jnp.float32),
        pltpu.VMEM((_GK, 2 * _D), jnp.float32),
        pltpu.VMEM((_G, 2 * _D), jnp.float32),
        pltpu.VMEM((_G, 2 * _D), jnp.float32),
        pltpu.VMEM((_GK, 2 * _D), jnp.float32),
        pltpu.VMEM((_D,), jnp.float32),
        pltpu.VMEM((_BPW,), jnp.float32),
        pltpu.VMEM((_BPW,), jnp.float32),
        pltpu.VMEM((_BPW * _K,), jnp.float32),
        pltpu.SemaphoreType.DMA,
        pltpu.SemaphoreType.DMA,
        pltpu.SemaphoreType.DMA,
    ],
)(_sc_body)


_PCOLS = 32768
_PGRID = -(-_VOCAB // _PCOLS)
_PROWS = _PGRID * _PCOLS // 2


def _pack_body(in_ref, out_ref):
    y = in_ref[...].T
    for j in range(_PCOLS // 256):
        lo = y[j * 256:j * 256 + 128]
        hi = y[j * 256 + 128:j * 256 + 256]
        out_ref[pl.ds(j * 128, 128), :] = jnp.concatenate([lo, hi], axis=1)


def _tc_pack(table_t):
    return pl.pallas_call(
        _pack_body,
        grid=(_PGRID,),
        in_specs=[pl.BlockSpec((_D, _PCOLS), lambda b: (0, b))],
        out_specs=pl.BlockSpec((_PCOLS // 2, 128), lambda b: (b, 0)),
        out_shape=jax.ShapeDtypeStruct((_PROWS, 128), jnp.float32),
    )(table_t)


def _tc_body(pos_ref, neg_ref, pred_ref, b_ref, loss_ref, fix_ref):
    pos = jnp.clip(pos_ref[...], -10.0, 10.0)
    neg = jnp.clip(neg_ref[...], -10.0, 10.0)
    sp_pos = jnp.maximum(-pos, 0.0) + jnp.log(1.0 + jnp.exp(-jnp.abs(pos)))
    sp_neg = jnp.maximum(neg, 0.0) + jnp.log(1.0 + jnp.exp(-jnp.abs(neg)))
    total = jnp.sum(sp_pos) + jnp.sum(sp_neg)
    loss_ref[0, 0] = total / _B
    fix_ref[...] = pred_ref[...] + b_ref[0, 0]


def _tc_finish(pos2d, neg2d, pred2d, b2d):
    return pl.pallas_call(
        _tc_body,
        out_shape=[
            jax.ShapeDtypeStruct((1, 1), jnp.float32),
            jax.ShapeDtypeStruct((_B // 128, 128), jnp.float32),
        ],
        in_specs=[
            pl.BlockSpec(memory_space=pltpu.VMEM),
            pl.BlockSpec(memory_space=pltpu.VMEM),
            pl.BlockSpec(memory_space=pltpu.VMEM),
            pl.BlockSpec(memory_space=pltpu.SMEM),
        ],
        out_specs=[
            pl.BlockSpec(memory_space=pltpu.SMEM),
            pl.BlockSpec(memory_space=pltpu.VMEM),
        ],
    )(pos2d, neg2d, pred2d, b2d)


def kernel(target_word, context_words, neg_words, u_table, v_table, W_dur, b_dur):
    tgt = target_word.astype(jnp.int32)
    ctx = context_words.astype(jnp.int32)
    negf = neg_words.astype(jnp.int32).reshape(_B * _K // 128, 128)
    w = W_dur.reshape(_D)
    u2 = _tc_pack(u_table.T)
    v2 = _tc_pack(v_table.T)
    pos, negdot, pred = _sc_dots(tgt, ctx, negf, u2, v2, w)
    loss, fix = _tc_finish(
        pos.reshape(_B // 128, 128),
        negdot.reshape(_B * _K // 128, 128),
        pred.reshape(_B // 128, 128),
        b_dur.reshape(1, 1),
    )
    return loss.reshape(()), fix.reshape(_B)
